# speedup vs baseline: 2.3447x; 2.3447x over previous
import functools

import jax
import jax.numpy as jnp
from jax import lax
from jax.experimental import pallas as pl
from jax.experimental.pallas import tpu as pltpu
from jax.experimental.pallas import tpu_sc as plsc

NC, NS, L = 2, 16, 16
NW = NC * NS
D = 1024
T = 2048
C = 8
VOCAB_PER_CB = 1024
K = 64
NCH = T // K

_mesh = plsc.VectorSubcoreMesh(core_axis_name="c", subcore_axis_name="s")


@functools.partial(
    pl.kernel,
    out_type=jax.ShapeDtypeStruct((NW * T, D), jnp.float32),
    mesh=_mesh,
    scratch_types=[
        pltpu.VMEM((T,), jnp.int32),
        pltpu.VMEM((K, D), jnp.float32),
        pltpu.SemaphoreType.DMA,
    ],
)
def _gather_kernel(table_hbm, ids_hbm, out_hbm, idx_v, rows_v, sem):
    wid = lax.axis_index("s") * NC + lax.axis_index("c")
    base = wid * T
    pltpu.sync_copy(ids_hbm.at[pl.ds(base, T)], idx_v)
    off = lax.rem(wid, C) * VOCAB_PER_CB

    def add_off(i, _):
        idx_v[pl.ds(i * L, L)] = idx_v[pl.ds(i * L, L)] + off
        return 0

    lax.fori_loop(0, T // L, add_off, 0)

    def chunk(g, _):
        pltpu.async_copy(
            table_hbm.at[idx_v.at[pl.ds(g * K, K)]], rows_v, sem
        ).wait()
        pltpu.sync_copy(rows_v, out_hbm.at[pl.ds(base + g * K, K)])
        return 0

    lax.fori_loop(0, NCH, chunk, 0)


def kernel(input_ids, table):
    b, c, t = input_ids.shape
    ids = input_ids.reshape(-1).astype(jnp.int32)
    out = _gather_kernel(table.astype(jnp.float32), ids)
    return out.reshape(b, c, t, D)


# speedup vs baseline: 2.5404x; 1.0834x over previous
import functools

import jax
import jax.numpy as jnp
from jax import lax
from jax.experimental import pallas as pl
from jax.experimental.pallas import tpu as pltpu
from jax.experimental.pallas import tpu_sc as plsc

NC, NS, L = 2, 16, 16
NW = NC * NS
D = 1024
T = 2048
C = 8
VOCAB_PER_CB = 1024
K = 16
NCH = T // K
NBUF = 4
NITER = NCH // NBUF

_mesh = plsc.VectorSubcoreMesh(core_axis_name="c", subcore_axis_name="s")


@functools.partial(
    pl.kernel,
    out_type=jax.ShapeDtypeStruct((NW * T, D), jnp.float32),
    mesh=_mesh,
    scratch_types=[
        pltpu.VMEM((T,), jnp.int32),
        [pltpu.VMEM((K, D), jnp.float32) for _ in range(NBUF)],
        [pltpu.SemaphoreType.DMA for _ in range(NBUF)],
        [pltpu.SemaphoreType.DMA for _ in range(NBUF)],
    ],
)
def _gather_kernel(table_hbm, ids_hbm, out_hbm, idx_v, rows, gsem, wsem):
    wid = lax.axis_index("s") * NC + lax.axis_index("c")
    base = wid * T
    pltpu.sync_copy(ids_hbm.at[pl.ds(base, T)], idx_v)
    off = lax.rem(wid, C) * VOCAB_PER_CB

    def add_off(i, _):
        idx_v[pl.ds(i * L, L)] = idx_v[pl.ds(i * L, L)] + off
        return 0

    lax.fori_loop(0, T // L, add_off, 0)

    def gather_start(i, b):
        pltpu.async_copy(table_hbm.at[idx_v.at[pl.ds(i * K, K)]], rows[b], gsem[b])

    def gather_wait(i, b):
        pltpu.make_async_copy(
            table_hbm.at[idx_v.at[pl.ds(i * K, K)]], rows[b], gsem[b]
        ).wait()

    def write_start(i, b):
        pltpu.async_copy(rows[b], out_hbm.at[pl.ds(base + i * K, K)], wsem[b])

    def write_wait(i, b):
        pltpu.make_async_copy(
            rows[b], out_hbm.at[pl.ds(base + i * K, K)], wsem[b]
        ).wait()

    for b in range(NBUF):
        gather_start(b, b)

    def body(j, _):
        for b in range(NBUF):
            i = j * NBUF + b
            gather_wait(i, b)
            write_start(i, b)

            @pl.when(j < NITER - 1)
            def _():
                write_wait(i, b)
                gather_start(i + NBUF, b)

        return 0

    lax.fori_loop(0, NITER, body, 0)

    for b in range(NBUF):
        write_wait((NITER - 1) * NBUF + b, b)


def kernel(input_ids, table):
    b, c, t = input_ids.shape
    ids = input_ids.reshape(-1).astype(jnp.int32)
    out = _gather_kernel(table.astype(jnp.float32), ids)
    return out.reshape(b, c, t, D)
